# baseline (speedup 1.0000x reference)
import functools

import jax
import jax.numpy as jnp
from jax import lax
from jax.experimental import pallas as pl
from jax.experimental.pallas import tpu as pltpu
from jax.experimental.pallas import tpu_sc as plsc

_TABLE_ROWS = 1_000_000
_D = 64
_DP = 128
_B = 819_200

_info = plsc.get_sparse_core_info()
_NC, _NS = _info.num_cores, _info.num_subcores
_NW = _NC * _NS
_BPW = _B // _NW
_CH = 256
_NCHUNK = _BPW // _CH

_mesh = plsc.VectorSubcoreMesh(core_axis_name="c", subcore_axis_name="s")


@functools.partial(
    pl.kernel,
    out_type=jax.ShapeDtypeStruct((_D, _B), jnp.float32),
    mesh=_mesh,
    scratch_types=[
        pltpu.VMEM((_BPW,), jnp.int32),
        pltpu.VMEM((2, _CH, _DP), jnp.float32),
        pltpu.VMEM((2, _D, _CH), jnp.float32),
        pltpu.SemaphoreType.DMA,
        pltpu.SemaphoreType.DMA,
        pltpu.SemaphoreType.DMA,
        pltpu.SemaphoreType.DMA,
    ],
    compiler_params=pltpu.CompilerParams(needs_layout_passes=False),
)
def _gather(table_hbm, idx_hbm, out_t_hbm, idx_v, rows_v, trows_v,
            gsem0, gsem1, wsem0, wsem1):
    wid = lax.axis_index("s") * _NC + lax.axis_index("c")
    base = wid * _BPW
    gsems = (gsem0, gsem1)
    wsems = (wsem0, wsem1)

    pltpu.sync_copy(idx_hbm.at[pl.ds(base, _BPW)], idx_v)

    lanes = lax.iota(jnp.int32, 16)
    row_idx = [lanes + 16 * q for q in range(4)]

    def fire_gather(g, b):
        return pltpu.async_copy(
            table_hbm.at[idx_v.at[pl.ds(g * _CH, _CH)]], rows_v.at[b], gsems[b]
        )

    def transpose_chunk(b):
        rows_ref = rows_v.at[b]
        trows_ref = trows_v.at[b]

        def body(r, _):
            col = lanes * 0 + r
            for q in range(4):
                v = rows_ref[r, pl.ds(16 * q, 16)]
                plsc.store_scatter(trows_ref, [row_idx[q], col], v)
            return 0

        lax.fori_loop(0, _CH, body, 0)

    gathers = [None, None]
    writes = [None, None]
    gathers[0] = fire_gather(0, 0)
    gathers[1] = fire_gather(1, 1)

    for g in range(_NCHUNK):
        b = g % 2
        gathers[b].wait()
        if g >= 2:
            writes[b].wait()
        transpose_chunk(b)
        if g + 2 < _NCHUNK:
            gathers[b] = fire_gather(g + 2, b)
        off = pl.multiple_of(base + g * _CH, _CH)
        writes[b] = pltpu.async_copy(
            trows_v.at[b], out_t_hbm.at[:, pl.ds(off, _CH)], wsems[b]
        )

    writes[0].wait()
    writes[1].wait()


@jax.jit
def kernel(input, index, _):
    tpad = jnp.pad(input, ((0, 0), (0, _DP - _D)))
    out_t = _gather(tpad, index.astype(jnp.int32))
    gathered = out_t.T
    return (input, index, gathered)


# speedup vs baseline: 1.4436x; 1.4436x over previous
import functools

import jax
import jax.numpy as jnp
from jax import lax
from jax.experimental import pallas as pl
from jax.experimental.pallas import tpu as pltpu
from jax.experimental.pallas import tpu_sc as plsc

_TABLE_ROWS = 1_000_000
_D = 64
_DP = 128
_B = 819_200

_info = plsc.get_sparse_core_info()
_NC, _NS = _info.num_cores, _info.num_subcores
_NW = _NC * _NS
_BPW = _B // _NW
_CH = 400
_NCHUNK = _BPW // _CH

_mesh = plsc.VectorSubcoreMesh(core_axis_name="c", subcore_axis_name="s")


@functools.partial(
    pl.kernel,
    out_type=jax.ShapeDtypeStruct((_B, _DP), jnp.float32),
    mesh=_mesh,
    scratch_types=[
        pltpu.VMEM((_BPW,), jnp.int32),
        pltpu.VMEM((2, _CH, _DP), jnp.float32),
        pltpu.SemaphoreType.DMA,
        pltpu.SemaphoreType.DMA,
        pltpu.SemaphoreType.DMA,
        pltpu.SemaphoreType.DMA,
    ],
)
def _gather(table_hbm, idx_hbm, out_hbm, idx_v, rows_v, gsem0, gsem1, wsem0, wsem1):
    wid = lax.axis_index("s") * _NC + lax.axis_index("c")
    base = wid * _BPW
    gsems = (gsem0, gsem1)
    wsems = (wsem0, wsem1)

    pltpu.sync_copy(idx_hbm.at[pl.ds(base, _BPW)], idx_v)

    gathers = [None, None]
    writes = [None, None]
    for g in range(2):
        gathers[g % 2] = pltpu.async_copy(
            table_hbm.at[idx_v.at[pl.ds(g * _CH, _CH)]], rows_v.at[g % 2], gsems[g % 2]
        )

    for g in range(_NCHUNK):
        b = g % 2
        gathers[b].wait()
        writes[b] = pltpu.async_copy(
            rows_v.at[b], out_hbm.at[pl.ds(base + g * _CH, _CH)], wsems[b]
        )
        if g + 2 < _NCHUNK:
            writes[b].wait()
            gathers[b] = pltpu.async_copy(
                table_hbm.at[idx_v.at[pl.ds((g + 2) * _CH, _CH)]],
                rows_v.at[b],
                gsems[b],
            )
    writes[(_NCHUNK - 2) % 2].wait()
    writes[(_NCHUNK - 1) % 2].wait()


@jax.jit
def kernel(input, index, _):
    tpad = jnp.pad(input, ((0, 0), (0, _DP - _D)))
    padded_out = _gather(tpad, index.astype(jnp.int32))
    gathered = padded_out[:, :_D]
    return (input, index, gathered)


# speedup vs baseline: 1.4443x; 1.0005x over previous
import functools

import jax
import jax.numpy as jnp
from jax import lax
from jax.experimental import pallas as pl
from jax.experimental.pallas import tpu as pltpu
from jax.experimental.pallas import tpu_sc as plsc

_TABLE_ROWS = 1_000_000
_D = 64
_DP = 128
_B = 819_200

_info = plsc.get_sparse_core_info()
_NC, _NS = _info.num_cores, _info.num_subcores
_NW = _NC * _NS
_BPW = _B // _NW
_CH = 256
_NCHUNK = _BPW // _CH

_mesh = plsc.VectorSubcoreMesh(core_axis_name="c", subcore_axis_name="s")


@functools.partial(
    pl.kernel,
    out_type=jax.ShapeDtypeStruct((_B, _DP), jnp.float32),
    mesh=_mesh,
    scratch_types=[
        pltpu.VMEM((_BPW,), jnp.int32),
        pltpu.VMEM((3, _CH, _DP), jnp.float32),
        pltpu.SemaphoreType.DMA,
        pltpu.SemaphoreType.DMA,
        pltpu.SemaphoreType.DMA,
        pltpu.SemaphoreType.DMA,
        pltpu.SemaphoreType.DMA,
        pltpu.SemaphoreType.DMA,
    ],
)
def _gather(table_hbm, idx_hbm, out_hbm, idx_v, rows_v,
            gsem0, gsem1, gsem2, wsem0, wsem1, wsem2):
    wid = lax.axis_index("s") * _NC + lax.axis_index("c")
    base = wid * _BPW
    gsems = (gsem0, gsem1, gsem2)
    wsems = (wsem0, wsem1, wsem2)

    pltpu.sync_copy(idx_hbm.at[pl.ds(base, _BPW)], idx_v)

    gathers = [None, None, None]
    writes = [None, None, None]
    for g in range(3):
        gathers[g % 3] = pltpu.async_copy(
            table_hbm.at[idx_v.at[pl.ds(g * _CH, _CH)]], rows_v.at[g % 3], gsems[g % 3]
        )

    for g in range(_NCHUNK):
        b = g % 3
        gathers[b].wait()
        writes[b] = pltpu.async_copy(
            rows_v.at[b], out_hbm.at[pl.ds(base + g * _CH, _CH)], wsems[b]
        )
        if g + 3 < _NCHUNK:
            writes[b].wait()
            gathers[b] = pltpu.async_copy(
                table_hbm.at[idx_v.at[pl.ds((g + 3) * _CH, _CH)]],
                rows_v.at[b],
                gsems[b],
            )
    for b in ((_NCHUNK - 3) % 3, (_NCHUNK - 2) % 3, (_NCHUNK - 1) % 3):
        writes[b].wait()


@jax.jit
def kernel(input, index, _):
    tpad = jnp.pad(input, ((0, 0), (0, _DP - _D)))
    padded_out = _gather(tpad, index.astype(jnp.int32))
    gathered = padded_out[:, :_D]
    return (input, index, gathered)
